# baseline (speedup 1.0000x reference)
import functools

import jax
import jax.numpy as jnp
from jax import lax
from jax.experimental import pallas as pl
from jax.experimental.pallas import tpu as pltpu
from jax.experimental.pallas import tpu_sc as plsc


@functools.lru_cache(maxsize=None)
def _build_gather(n, v, d):
    info = plsc.get_sparse_core_info()
    nw = info.num_cores * info.num_subcores
    assert n % nw == 0
    b_per_w = n // nw
    C = 512
    assert b_per_w % C == 0
    n_chunks = b_per_w // C

    mesh = plsc.VectorSubcoreMesh(core_axis_name="c", subcore_axis_name="s")

    @functools.partial(
        pl.kernel,
        mesh=mesh,
        out_type=jax.ShapeDtypeStruct((n, d), jnp.float32),
        scratch_types=[
            pltpu.VMEM((C,), jnp.int32),
            pltpu.VMEM((C, d), jnp.float32),
            pltpu.SemaphoreType.DMA,
        ],
        compiler_params=pltpu.CompilerParams(use_tc_tiling_on_sc=False),
    )
    def k(table_hbm, ids_hbm, out_hbm, idx_v, rows_v, sem):
        wid = lax.axis_index("s") * info.num_cores + lax.axis_index("c")
        base = wid * b_per_w

        def body(i, carry):
            off = base + i * C
            pltpu.sync_copy(ids_hbm.at[pl.ds(off, C)], idx_v)
            pltpu.async_copy(table_hbm.at[idx_v], rows_v, sem).wait()
            pltpu.sync_copy(rows_v, out_hbm.at[pl.ds(off, C)])
            return carry

        lax.fori_loop(0, n_chunks, body, 0)

    return k


def kernel(input_ids, table):
    b, s = input_ids.shape
    v, d = table.shape
    n = b * s
    ids = input_ids.reshape(n).astype(jnp.int32)
    out = _build_gather(n, v, d)(table, ids)
    return (out.reshape(b, s, d),)


# speedup vs baseline: 1.0443x; 1.0443x over previous
import functools

import jax
import jax.numpy as jnp
from jax import lax
from jax.experimental import pallas as pl
from jax.experimental.pallas import tpu as pltpu
from jax.experimental.pallas import tpu_sc as plsc


@functools.lru_cache(maxsize=None)
def _build_gather(n, v, d):
    info = plsc.get_sparse_core_info()
    nw = info.num_cores * info.num_subcores
    assert n % nw == 0
    b_per_w = n // nw
    C = 512
    assert b_per_w % C == 0
    n_chunks = b_per_w // C
    assert n_chunks % 2 == 0 and n_chunks >= 4

    mesh = plsc.VectorSubcoreMesh(core_axis_name="c", subcore_axis_name="s")

    @functools.partial(
        pl.kernel,
        mesh=mesh,
        out_type=jax.ShapeDtypeStruct((n, d), jnp.float32),
        scratch_types=[
            pltpu.VMEM((2, C), jnp.int32),
            pltpu.VMEM((2, C, d), jnp.float32),
            pltpu.SemaphoreType.DMA,
            pltpu.SemaphoreType.DMA,
            pltpu.SemaphoreType.DMA,
            pltpu.SemaphoreType.DMA,
            pltpu.SemaphoreType.DMA,
            pltpu.SemaphoreType.DMA,
        ],
        compiler_params=pltpu.CompilerParams(use_tc_tiling_on_sc=False),
    )
    def k(table_hbm, ids_hbm, out_hbm, idx_v, rows_v,
          sem_i0, sem_i1, sem_g0, sem_g1, sem_s0, sem_s1):
        sem_i = (sem_i0, sem_i1)
        sem_g = (sem_g0, sem_g1)
        sem_s = (sem_s0, sem_s1)
        wid = lax.axis_index("s") * info.num_cores + lax.axis_index("c")
        base = wid * b_per_w

        def idx_copy(i, b):
            return pltpu.make_async_copy(
                ids_hbm.at[pl.ds(base + i * C, C)], idx_v.at[b], sem_i[b])

        def gather_copy(b):
            return pltpu.make_async_copy(
                table_hbm.at[idx_v.at[b]], rows_v.at[b], sem_g[b])

        def store_copy(i, b):
            return pltpu.make_async_copy(
                rows_v.at[b], out_hbm.at[pl.ds(base + i * C, C)], sem_s[b])

        idx_copy(0, 0).start()
        idx_copy(1, 1).start()
        idx_copy(0, 0).wait()
        gather_copy(0).start()

        def body(g, carry):
            for b in (0, 1):
                i = 2 * g + b
                ob = 1 - b

                @pl.when(i + 1 < n_chunks)
                def _():
                    idx_copy(i + 1, ob).wait()

                @pl.when(i >= 1)
                def _():
                    store_copy(i - 1, ob).wait()

                @pl.when(i + 1 < n_chunks)
                def _():
                    gather_copy(ob).start()

                gather_copy(b).wait()

                @pl.when(i + 2 < n_chunks)
                def _():
                    idx_copy(i + 2, b).start()

                store_copy(i, b).start()
            return carry

        lax.fori_loop(0, n_chunks // 2, body, 0)
        store_copy(n_chunks - 1, (n_chunks - 1) % 2).wait()

    return k


def kernel(input_ids, table):
    b, s = input_ids.shape
    v, d = table.shape
    n = b * s
    ids = input_ids.reshape(n).astype(jnp.int32)
    out = _build_gather(n, v, d)(table, ids)
    return (out.reshape(b, s, d),)


# speedup vs baseline: 1.0455x; 1.0012x over previous
import functools

import jax
import jax.numpy as jnp
from jax import lax
from jax.experimental import pallas as pl
from jax.experimental.pallas import tpu as pltpu
from jax.experimental.pallas import tpu_sc as plsc


@functools.lru_cache(maxsize=None)
def _build_gather(n, v, d):
    info = plsc.get_sparse_core_info()
    nw = info.num_cores * info.num_subcores
    assert n % nw == 0
    b_per_w = n // nw
    C = 512
    assert b_per_w % C == 0
    n_chunks = b_per_w // C
    assert n_chunks % 2 == 0 and n_chunks >= 4

    mesh = plsc.VectorSubcoreMesh(core_axis_name="c", subcore_axis_name="s")

    @functools.partial(
        pl.kernel,
        mesh=mesh,
        out_type=jax.ShapeDtypeStruct((n, d), jnp.float32),
        scratch_types=[
            pltpu.VMEM((2, C), jnp.int32),
            pltpu.VMEM((2, C, d), jnp.float32),
            pltpu.SemaphoreType.DMA,
            pltpu.SemaphoreType.DMA,
            pltpu.SemaphoreType.DMA,
            pltpu.SemaphoreType.DMA,
            pltpu.SemaphoreType.DMA,
            pltpu.SemaphoreType.DMA,
        ],
        compiler_params=pltpu.CompilerParams(use_tc_tiling_on_sc=False),
    )
    def k(table_hbm, ids_hbm, out_hbm, idx_v, rows_v,
          sem_i0, sem_i1, sem_g0, sem_g1, sem_s0, sem_s1):
        sem_i = (sem_i0, sem_i1)
        sem_g = (sem_g0, sem_g1)
        sem_s = (sem_s0, sem_s1)
        wid = lax.axis_index("s") * info.num_cores + lax.axis_index("c")
        base = wid * b_per_w

        def idx_copy(i, b):
            return pltpu.make_async_copy(
                ids_hbm.at[pl.ds(base + i * C, C)], idx_v.at[b], sem_i[b])

        K = 8
        S = C // K

        def gather_start(b):
            for j in range(K):
                pltpu.make_async_copy(
                    table_hbm.at[idx_v.at[b, pl.ds(j * S, S)]],
                    rows_v.at[b, pl.ds(j * S, S)],
                    sem_g[b]).start()

        def gather_wait(b):
            for j in range(K):
                pltpu.make_async_copy(
                    table_hbm.at[idx_v.at[b, pl.ds(j * S, S)]],
                    rows_v.at[b, pl.ds(j * S, S)],
                    sem_g[b]).wait()

        def store_copy(i, b):
            return pltpu.make_async_copy(
                rows_v.at[b], out_hbm.at[pl.ds(base + i * C, C)], sem_s[b])

        idx_copy(0, 0).start()
        idx_copy(1, 1).start()
        idx_copy(0, 0).wait()
        gather_start(0)

        def body(g, carry):
            for b in (0, 1):
                i = 2 * g + b
                ob = 1 - b

                @pl.when(i + 1 < n_chunks)
                def _():
                    idx_copy(i + 1, ob).wait()

                @pl.when(i >= 1)
                def _():
                    store_copy(i - 1, ob).wait()

                @pl.when(i + 1 < n_chunks)
                def _():
                    gather_start(ob)

                gather_wait(b)

                @pl.when(i + 2 < n_chunks)
                def _():
                    idx_copy(i + 2, b).start()

                store_copy(i, b).start()
            return carry

        lax.fori_loop(0, n_chunks // 2, body, 0)
        store_copy(n_chunks - 1, (n_chunks - 1) % 2).wait()

    return k


def kernel(input_ids, table):
    b, s = input_ids.shape
    v, d = table.shape
    n = b * s
    ids = input_ids.reshape(n).astype(jnp.int32)
    out = _build_gather(n, v, d)(table, ids)
    return (out.reshape(b, s, d),)
